# Initial kernel scaffold; baseline (speedup 1.0000x reference)
import jax
import jax.numpy as jnp
from jax.experimental import pallas as pl


def kernel(gold, scores, nbins, embed_weight):
    raise NotImplementedError("write your pallas kernel here")


# speedup vs baseline: 6.4676x; 6.4676x over previous
import functools

import jax
import jax.numpy as jnp
from jax import lax
from jax.experimental import pallas as pl
from jax.experimental.pallas import tpu as pltpu
from jax.experimental.pallas import tpu_sc as plsc

_LANES = 16
_NUM_WORKERS = 32


def _sc_hash_gather(gold_flat, emb_flat, scale_vec, nbins_vec):
    n = gold_flat.shape[0]
    v4 = emb_flat.shape[0]
    v = v4 // 4
    per_w = n // _NUM_WORKERS
    mesh = plsc.VectorSubcoreMesh(core_axis_name="c", subcore_axis_name="s")

    @functools.partial(
        pl.kernel,
        mesh=mesh,
        compiler_params=pltpu.CompilerParams(needs_layout_passes=False),
        out_type=jax.ShapeDtypeStruct((n,), jnp.int32),
        scratch_types=[
            pltpu.VMEM((v4,), jnp.float32),
            pltpu.VMEM((v,), jnp.int32),
            pltpu.VMEM((per_w,), jnp.int32),
            pltpu.VMEM((per_w,), jnp.int32),
            pltpu.VMEM((_LANES,), jnp.float32),
            pltpu.VMEM((_LANES,), jnp.int32),
            pltpu.SemaphoreType.DMA,
        ],
    )
    def sc_kernel(gold_hbm, emb_hbm, scale_hbm, nb_hbm, h_hbm,
                  emb_v, tbl_v, gold_v, h_v, scale_v, nb_v, sem):
        wid = lax.axis_index("s") * 2 + lax.axis_index("c")
        base = wid * per_w
        gold_dma = pltpu.async_copy(gold_hbm.at[pl.ds(base, per_w)], gold_v, sem)
        pltpu.sync_copy(emb_hbm, emb_v)
        pltpu.sync_copy(scale_hbm, scale_v)
        pltpu.sync_copy(nb_hbm, nb_v)
        scale = scale_v[...]
        nb = nb_v[...]
        nb2 = nb * nb
        lane = lax.iota(jnp.int32, _LANES)

        def hash_body(i, carry):
            r = i * _LANES
            i0 = (r + lane) * 4
            x0 = plsc.load_gather(emb_v, [i0])
            x1 = plsc.load_gather(emb_v, [i0 + 1])
            x2 = plsc.load_gather(emb_v, [i0 + 2])
            x3 = plsc.load_gather(emb_v, [i0 + 3])
            q0 = (x0 * scale).astype(jnp.int32)
            q1 = (x1 * scale).astype(jnp.int32)
            q2 = (x2 * scale).astype(jnp.int32)
            hv = jnp.where(x3 < 0.02, 0, 1 + q0 * nb2 + q1 * nb + q2)
            tbl_v[pl.ds(r, _LANES)] = hv
            return carry

        lax.fori_loop(0, v // _LANES, hash_body, 0)
        gold_dma.wait()

        def gather_body(i, carry):
            r = i * _LANES
            g = gold_v[pl.ds(r, _LANES)]
            h_v[pl.ds(r, _LANES)] = plsc.load_gather(tbl_v, [g])
            return carry

        lax.fori_loop(0, per_w // _LANES, gather_body, 0)
        pltpu.sync_copy(h_v, h_hbm.at[pl.ds(base, per_w)])

    return sc_kernel(gold_flat, emb_flat, scale_vec, nbins_vec)


def _tc_nll_sum(scores3, h3):
    b, c, s = scores3.shape
    blk = 4096
    j_steps = s // blk

    def body(x_ref, h_ref, o_ref):
        x = x_ref[0]
        m = jnp.max(x, axis=0, keepdims=True)
        lse = m + jnp.log(jnp.sum(jnp.exp(x - m), axis=0, keepdims=True))
        hh = h_ref[0]
        cid = lax.broadcasted_iota(jnp.int32, (c, blk), 0)
        pick = jnp.sum(jnp.where(cid == hh, x, 0.0), axis=0, keepdims=True)
        part = jnp.sum(lse - pick)

        @pl.when((pl.program_id(0) == 0) & (pl.program_id(1) == 0))
        def _init():
            o_ref[0, 0] = 0.0

        o_ref[0, 0] += part

    out = pl.pallas_call(
        body,
        grid=(b, j_steps),
        in_specs=[
            pl.BlockSpec((1, c, blk), lambda bi, ji: (bi, 0, ji)),
            pl.BlockSpec((1, 1, blk), lambda bi, ji: (bi, 0, ji)),
        ],
        out_specs=pl.BlockSpec(memory_space=pltpu.SMEM),
        out_shape=jax.ShapeDtypeStruct((1, 1), jnp.float32),
    )(scores3, h3)
    return out[0, 0]


def kernel(gold, scores, nbins, embed_weight):
    b, c = scores.shape[0], scores.shape[1]
    s = scores.shape[2] * scores.shape[3] * scores.shape[4]
    n = gold.size
    scale_vec = jnp.full((_LANES,), nbins - jnp.float32(0.001), jnp.float32)
    nbins_vec = jnp.full((_LANES,), nbins, jnp.int32)
    h = _sc_hash_gather(gold.reshape(-1), embed_weight.reshape(-1),
                        scale_vec, nbins_vec)
    total = _tc_nll_sum(scores.reshape(b, c, s), h.reshape(b, 1, s))
    return total / n
